# baseline (speedup 1.0000x reference)
import functools

import jax
import jax.numpy as jnp
from jax import lax
from jax.experimental import pallas as pl
from jax.experimental.pallas import tpu as pltpu
from jax.experimental.pallas import tpu_sc as plsc

NC = 2
NS = 16
NW = NC * NS
LANES = 16


@functools.partial(jax.jit, static_argnums=(4, 5, 6))
def _sc_embed(inp_half, inp_off, token2, pos_block, b, l, h):
    rb = b // NW
    half = l // 2
    hc = h // LANES
    h2 = 2 * h

    mesh = plsc.VectorSubcoreMesh(core_axis_name="c", subcore_axis_name="s")

    def body(inp_hbm, off_hbm, tok_hbm, pos_hbm, out_hbm, idx_v, off_s,
             rows_v, out_v, pos_v, sem):
        wid = lax.axis_index("s") * NC + lax.axis_index("c")
        pltpu.sync_copy(pos_hbm, pos_v)

        def row_body(i, _):
            bi = wid * rb + i
            pltpu.sync_copy(inp_hbm.at[bi], idx_v)
            pltpu.sync_copy(off_hbm.at[bi], off_s.at[pl.ds(0, l)])
            cp0 = pltpu.async_copy(
                tok_hbm.at[idx_v.at[0]], rows_v.at[pl.ds(0, half)], sem)
            cp1 = pltpu.async_copy(
                tok_hbm.at[idx_v.at[1]], rows_v.at[pl.ds(half, half)], sem)
            cp0.wait()
            cp1.wait()

            def tok_body(t, _):
                off_vec = off_s[pl.ds(2 * t, LANES)]
                off0 = off_vec[0]
                off1 = off_vec[1]
                for c in range(hc):
                    s0 = rows_v[2 * t, pl.ds(off0 + c * LANES, LANES)]
                    out_v[t, pl.ds(c * LANES, LANES)] = (
                        s0 + pos_v[2 * t, pl.ds(c * LANES, LANES)])
                    s1 = rows_v[2 * t + 1, pl.ds(off1 + c * LANES, LANES)]
                    out_v[t, pl.ds(h + c * LANES, LANES)] = (
                        s1 + pos_v[2 * t + 1, pl.ds(c * LANES, LANES)])
                return ()

            lax.fori_loop(0, half, tok_body, ())
            pltpu.sync_copy(out_v, out_hbm.at[bi])
            return ()

        lax.fori_loop(0, rb, row_body, ())

    call = pl.kernel(
        body,
        out_type=jax.ShapeDtypeStruct((b, l // 2, h2), jnp.float32),
        mesh=mesh,
        scratch_types=[
            pltpu.VMEM((2, half), jnp.int32),
            pltpu.VMEM((l + LANES,), jnp.int32),
            pltpu.VMEM((l, h2), jnp.float32),
            pltpu.VMEM((half, h2), jnp.float32),
            pltpu.VMEM((l, h), jnp.float32),
            pltpu.SemaphoreType.DMA,
        ],
        compiler_params=pltpu.CompilerParams(use_tc_tiling_on_sc=False),
    )
    return call(inp_half, inp_off, token2, pos_block)


def kernel(input, token_table, pos_table):
    b, l = input.shape
    h = token_table.shape[1]
    inp_half = (input >> 1).reshape(b, 2, l // 2)
    inp_off = (input & 1) * h
    token2 = token_table.reshape(token_table.shape[0] // 2, 2 * h)
    pos_block = lax.slice(pos_table, (1, 0), (1 + l, h))
    out = _sc_embed(inp_half, inp_off, token2, pos_block, b, l, h)
    return out.reshape(b, l, h)


# speedup vs baseline: 1.4109x; 1.4109x over previous
import functools

import jax
import jax.numpy as jnp
from jax import lax
from jax.experimental import pallas as pl
from jax.experimental.pallas import tpu as pltpu
from jax.experimental.pallas import tpu_sc as plsc

NC = 2
NS = 16
NW = NC * NS
LANES = 16


@functools.partial(jax.jit, static_argnums=(3, 4, 5))
def _sc_embed(inp2, token_table, pos_block, b, l, h):
    rb = b // NW
    half = l // 2
    hc = h // LANES
    h2 = 2 * h

    mesh = plsc.VectorSubcoreMesh(core_axis_name="c", subcore_axis_name="s")

    def body(inp_hbm, tok_hbm, pos_hbm, out_hbm,
             idx0, idx1, rows0, rows1, outv0, outv1, pos_v,
             sem_i, sem_g, sem_o):
        wid = lax.axis_index("s") * NC + lax.axis_index("c")
        base = wid * rb
        pltpu.sync_copy(pos_hbm, pos_v)

        idx_v = (idx0, idx1)
        rows_v = (rows0, rows1)
        out_v = (outv0, outv1)

        def issue_idx(i, s):
            return pltpu.async_copy(inp_hbm.at[base + i], idx_v[s], sem_i)

        def issue_gather(i, s):
            pltpu.async_copy(
                tok_hbm.at[idx_v[s].at[0]], rows_v[s].at[pl.ds(0, half)],
                sem_g)
            pltpu.async_copy(
                tok_hbm.at[idx_v[s].at[1]], rows_v[s].at[pl.ds(half, half)],
                sem_g)

        def wait_gather(s):
            pltpu.make_async_copy(
                tok_hbm.at[idx_v[s].at[0]], rows_v[s].at[pl.ds(0, half)],
                sem_g).wait()
            pltpu.make_async_copy(
                tok_hbm.at[idx_v[s].at[1]], rows_v[s].at[pl.ds(half, half)],
                sem_g).wait()

        def wait_idx(s):
            pltpu.make_async_copy(inp_hbm.at[base], idx_v[s], sem_i).wait()

        def compute(i, s):
            def tok_body(t, _):
                for c in range(hc):
                    sl = pl.ds(c * LANES, LANES)
                    out_v[s][t, pl.ds(c * LANES, LANES)] = (
                        rows_v[s][2 * t, sl] + pos_v[2 * t, sl])
                    out_v[s][t, pl.ds(h + c * LANES, LANES)] = (
                        rows_v[s][2 * t + 1, sl] + pos_v[2 * t + 1, sl])
                return ()
            lax.fori_loop(0, half, tok_body, (), unroll=2)

        def issue_out(i, s):
            pltpu.async_copy(out_v[s], out_hbm.at[base + i], sem_o)

        def wait_out(i, s):
            pltpu.make_async_copy(out_v[s], out_hbm.at[base + i], sem_o).wait()

        issue_idx(0, 0).wait()
        issue_gather(0, 0)
        issue_idx(1, 1)

        wait_gather(0)
        issue_idx(2, 0)
        wait_idx(1)
        issue_gather(1, 1)
        compute(0, 0)
        issue_out(0, 0)
        wait_gather(1)
        issue_idx(3, 1)
        wait_idx(0)
        issue_gather(2, 0)
        compute(1, 1)
        issue_out(1, 1)

        def pair_body(j, _):
            i0 = 2 * j

            def step(i, s):
                wait_gather(s)

                @pl.when(i + 2 < rb)
                def _():
                    issue_idx(i + 2, s)

                @pl.when(i + 1 < rb)
                def _():
                    wait_idx(1 - s)
                    issue_gather(i + 1, 1 - s)

                wait_out(i - 2, s)
                compute(i, s)
                issue_out(i, s)

            step(i0, 0)
            step(i0 + 1, 1)
            return ()

        lax.fori_loop(1, rb // 2, pair_body, ())

        wait_out(rb - 2, 0)
        wait_out(rb - 1, 1)

    call = pl.kernel(
        body,
        out_type=jax.ShapeDtypeStruct((b, half, h2), jnp.float32),
        mesh=mesh,
        scratch_types=[
            pltpu.VMEM((2, half), jnp.int32),
            pltpu.VMEM((2, half), jnp.int32),
            pltpu.VMEM((l, h), jnp.float32),
            pltpu.VMEM((l, h), jnp.float32),
            pltpu.VMEM((half, h2), jnp.float32),
            pltpu.VMEM((half, h2), jnp.float32),
            pltpu.VMEM((l, h), jnp.float32),
            pltpu.SemaphoreType.DMA,
            pltpu.SemaphoreType.DMA,
            pltpu.SemaphoreType.DMA,
        ],
        compiler_params=pltpu.CompilerParams(use_tc_tiling_on_sc=False),
    )
    return call(inp2, token_table, pos_block)


def kernel(input, token_table, pos_table):
    b, l = input.shape
    h = token_table.shape[1]
    inp2 = input.reshape(b, 2, l // 2)
    pos_block = lax.slice(pos_table, (1, 0), (1 + l, h))
    out = _sc_embed(inp2, token_table, pos_block, b, l, h)
    return out.reshape(b, l, h)
